# baseline (speedup 1.0000x reference)
import jax
import jax.numpy as jnp
from jax import lax
from jax.experimental import pallas as pl
from jax.experimental.pallas import tpu as pltpu
from jax.experimental.pallas import tpu_sc as plsc

POSE_DIM = 69
RH_DIM = 3
CP = 4
BATCH = 4096
BLUR = 16
IMG_NUM = 100000

NC = 2
NS = 16
NW = NC * NS
BPW = BATCH // NW
POSE_CHUNKS = (0, 16, 32, 33, 49, 53)
SROW = RH_DIM * CP
SMALL2 = 2 * SROW
SMALL_VROWS = IMG_NUM * SMALL2 // 128
BT = BATCH * BLUR


def _blend_kernel(idx_hbm, coef_hbm, poses_hbm, small_hbm,
                  pose_out, rh_out, th_out,
                  idx_v, idxa_v, poses_g, tail_g, small_g, coef_v,
                  pose_buf, small_buf, semA, semB):
    wid = lax.axis_index("s") * NC + lax.axis_index("c")
    base = wid * BPW
    iota = lax.broadcasted_iota(jnp.int32, (16,), 0)

    pltpu.sync_copy(idx_hbm.at[pl.ds(base, BPW)], idx_v)

    def vidx_body(j, carry):
        idxc = idx_v[pl.ds(j * 16, 16)]
        a0 = lax.shift_right_logical(idxc * SMALL2, 7)
        idxa_v[0, pl.ds(j * 16, 16)] = a0
        idxa_v[1, pl.ds(j * 16, 16)] = jnp.minimum(a0 + 1, SMALL_VROWS - 1)
        return carry
    lax.fori_loop(0, BPW // 16, vidx_body, 0)

    tail_start = pl.multiple_of(jnp.asarray(256, jnp.int32), 128)
    gA = [pltpu.async_copy(poses_hbm.at[idx_v, pl.ds(0, 128)],
                           poses_g.at[0], semA),
          pltpu.async_copy(poses_hbm.at[idx_v, pl.ds(128, 128)],
                           poses_g.at[1], semA),
          pltpu.async_copy(poses_hbm.at[idx_v, pl.ds(tail_start, 128)],
                           tail_g, semA)]
    gB = [pltpu.async_copy(small_hbm.at[idxa_v.at[k]], small_g.at[k], semB)
          for k in range(2)]

    pltpu.sync_copy(coef_hbm, coef_v)

    def c(i, t):
        return coef_v[pl.ds((i * BLUR + t) * 16, 16)]

    for g in gA:
        g.wait()

    def bc_body(bc, carry):
        b0 = bc * 8

        def pose_body(b, inner):
            row = b0 + b
            rowv = jnp.full((16,), row, jnp.int32)
            v = []
            for d0 in POSE_CHUNKS:
                vi = []
                for i in range(CP):
                    e0 = i * POSE_DIM + d0
                    if e0 >= 256:
                        vi.append(tail_g[row, pl.ds(e0 - 256, 16)])
                    elif e0 + 15 < 128:
                        vi.append(poses_g[0, row, pl.ds(e0, 16)])
                    elif e0 >= 128:
                        vi.append(poses_g[1, row, pl.ds(e0 - 128, 16)])
                    else:
                        e = e0 + iota
                        seg = (e >= 128).astype(jnp.int32)
                        vi.append(plsc.load_gather(
                            poses_g, [seg, rowv, e - 128 * seg]))
                v.append(vi)
            for t in range(BLUR):
                ct = [c(i, t) for i in range(CP)]
                for k in range(len(POSE_CHUNKS)):
                    acc = (v[k][0] * ct[0] + v[k][1] * ct[1]
                           + v[k][2] * ct[2] + v[k][3] * ct[3])
                    pose_buf[b, t, pl.ds(POSE_CHUNKS[k], 16)] = acc
            return inner
        lax.fori_loop(0, 8, pose_body, 0)

        pltpu.sync_copy(pose_buf, pose_out.at[pl.ds(base + b0, 8)])
        return carry
    lax.fori_loop(0, BPW // 8, bc_body, 0)

    for g in gB:
        g.wait()

    def small_pass(out_ref, qbase):
        def sbc_body(bc, carry):
            b0 = bc * 8
            rows = b0 + (iota & 7)
            idxs = plsc.load_gather(idx_v, [rows])
            ph = (idxs * SMALL2) & 127
            for d in range(RH_DIM):
                v = []
                for i in range(CP):
                    k = ph + (qbase + i * RH_DIM + d)
                    v.append(plsc.load_gather(
                        small_g, [lax.shift_right_logical(k, 7), rows,
                                  k & 127]))
                for t in range(BLUR):
                    acc = (v[0] * c(0, t) + v[1] * c(1, t)
                           + v[2] * c(2, t) + v[3] * c(3, t))
                    plsc.store_scatter(
                        small_buf, [iota & 7,
                                    jnp.full((16,), t, jnp.int32),
                                    jnp.full((16,), d, jnp.int32)], acc)
            pltpu.sync_copy(small_buf, out_ref.at[pl.ds(base + b0, 8)])
            return carry
        lax.fori_loop(0, BPW // 8, sbc_body, 0)

    small_pass(rh_out, 0)
    small_pass(th_out, SROW)


def _run(indices, coefF, poses_w, small2):
    f32 = jnp.float32
    kern = pl.kernel(
        _blend_kernel,
        out_type=[
            jax.ShapeDtypeStruct((BATCH, BLUR, POSE_DIM), f32),
            jax.ShapeDtypeStruct((BATCH, BLUR, RH_DIM), f32),
            jax.ShapeDtypeStruct((BATCH, BLUR, RH_DIM), f32),
        ],
        mesh=plsc.VectorSubcoreMesh(core_axis_name="c", subcore_axis_name="s",
                                    num_cores=NC, num_subcores=NS),
        compiler_params=pltpu.CompilerParams(use_tc_tiling_on_sc=True,
                                             needs_layout_passes=False),
        scratch_types=[
            pltpu.VMEM((BPW,), jnp.int32),
            pltpu.VMEM((2, BPW), jnp.int32),
            pltpu.VMEM((2, BPW, 128), f32),
            pltpu.VMEM((BPW, 128), f32),
            pltpu.VMEM((2, BPW, 128), f32),
            pltpu.VMEM((CP * BLUR * 16,), f32),
            pltpu.VMEM((8, BLUR, POSE_DIM), f32),
            pltpu.VMEM((8, BLUR, RH_DIM), f32),
            pltpu.SemaphoreType.DMA,
            pltpu.SemaphoreType.DMA,
        ],
    )
    return kern(indices, coefF, poses_w, small2)


def kernel(indices, blur_num, shapes_w, poses_w, Rhs_w, Ths_w, M):
    f32 = jnp.float32
    indices = indices.astype(jnp.int32)
    t = jnp.arange(BLUR, dtype=f32) / (jnp.asarray(blur_num, f32) - 1.0)
    t = jnp.where(t == 0.0, t + 1e-06, t)
    t = jnp.where(t == 1.0, t - 1e-06, t)
    tm = jnp.stack([jnp.ones_like(t), t, t ** 2, t ** 3], axis=-1)
    coeffs = tm @ M.astype(f32)
    coefF = jnp.broadcast_to(coeffs.T.reshape(CP * BLUR, 1),
                             (CP * BLUR, 16)).reshape(CP * BLUR * 16)

    small2 = jnp.concatenate(
        [Rhs_w.astype(f32), Ths_w.astype(f32)], axis=1)
    small2 = small2.reshape(SMALL_VROWS, 128)

    pose_t, rh_t, th_t = _run(indices, coefF, poses_w.astype(f32), small2)
    shape_t = jnp.broadcast_to(shapes_w.reshape(1, 1, 10).astype(f32),
                               (BATCH, BLUR, 10))
    return (shape_t, pose_t, rh_t, th_t)


# speedup vs baseline: 1.0561x; 1.0561x over previous
import jax
import jax.numpy as jnp
from jax import lax
from jax.experimental import pallas as pl
from jax.experimental.pallas import tpu as pltpu
from jax.experimental.pallas import tpu_sc as plsc

POSE_DIM = 69
RH_DIM = 3
CP = 4
BATCH = 4096
BLUR = 16
IMG_NUM = 100000

NC = 2
NS = 16
NW = NC * NS
BPW = BATCH // NW
POSE_CHUNKS = (0, 16, 32, 33, 49, 53)
SROW = RH_DIM * CP
SMALL2 = 2 * SROW
SMALL_VROWS = IMG_NUM * SMALL2 // 128
BT = BATCH * BLUR


def _blend_kernel(idx_hbm, coef_hbm, poses_hbm, small_hbm,
                  pose_out, rh_out, th_out,
                  idx_v, idxa_v, poses_g, tail_g, small_g, coef_v,
                  pose_buf, small_buf, semA, semB):
    wid = lax.axis_index("s") * NC + lax.axis_index("c")
    base = wid * BPW
    iota = lax.broadcasted_iota(jnp.int32, (16,), 0)

    pltpu.sync_copy(idx_hbm.at[pl.ds(base, BPW)], idx_v)

    def vidx_body(j, carry):
        idxc = idx_v[pl.ds(j * 16, 16)]
        a0 = lax.shift_right_logical(idxc * SMALL2, 7)
        idxa_v[0, pl.ds(j * 16, 16)] = a0
        idxa_v[1, pl.ds(j * 16, 16)] = jnp.minimum(a0 + 1, SMALL_VROWS - 1)
        return carry
    lax.fori_loop(0, BPW // 16, vidx_body, 0)

    tail_start = pl.multiple_of(jnp.asarray(256, jnp.int32), 128)
    gA = [pltpu.async_copy(poses_hbm.at[idx_v, pl.ds(0, 128)],
                           poses_g.at[0], semA),
          pltpu.async_copy(poses_hbm.at[idx_v, pl.ds(128, 128)],
                           poses_g.at[1], semA),
          pltpu.async_copy(poses_hbm.at[idx_v, pl.ds(tail_start, 128)],
                           tail_g, semA)]
    gB = [pltpu.async_copy(small_hbm.at[idxa_v.at[k]], small_g.at[k], semB)
          for k in range(2)]

    pltpu.sync_copy(coef_hbm, coef_v)

    def c(i, t):
        return coef_v[pl.ds((i * BLUR + t) * 16, 16)]

    for g in gA:
        g.wait()

    def bc_body(bc, carry):
        b0 = bc * 8

        def pose_body(b, inner):
            row = b0 + b
            rowv = jnp.full((16,), row, jnp.int32)
            v = []
            for d0 in POSE_CHUNKS:
                vi = []
                for i in range(CP):
                    e0 = i * POSE_DIM + d0
                    if e0 >= 256:
                        vi.append(tail_g[row, pl.ds(e0 - 256, 16)])
                    elif e0 + 15 < 128:
                        vi.append(poses_g[0, row, pl.ds(e0, 16)])
                    elif e0 >= 128:
                        vi.append(poses_g[1, row, pl.ds(e0 - 128, 16)])
                    else:
                        e = e0 + iota
                        seg = (e >= 128).astype(jnp.int32)
                        vi.append(plsc.load_gather(
                            poses_g, [seg, rowv, e - 128 * seg]))
                v.append(vi)
            for t in range(BLUR):
                ct = [c(i, t) for i in range(CP)]
                for k in range(len(POSE_CHUNKS)):
                    acc = (v[k][0] * ct[0] + v[k][1] * ct[1]
                           + v[k][2] * ct[2] + v[k][3] * ct[3])
                    pose_buf[b * BLUR + t, pl.ds(POSE_CHUNKS[k], 16)] = acc
            return inner
        lax.fori_loop(0, 8, pose_body, 0)

        pltpu.sync_copy(pose_buf,
                        pose_out.at[pl.ds((base + b0) * BLUR, 8 * BLUR)])
        return carry
    lax.fori_loop(0, BPW // 8, bc_body, 0)

    for g in gB:
        g.wait()

    def small_pass(out_ref, qbase):
        def sbc_body(bc, carry):
            b0 = bc * 8
            rows = b0 + (iota & 7)
            idxs = plsc.load_gather(idx_v, [rows])
            ph = (idxs * SMALL2) & 127
            for d in range(RH_DIM):
                v = []
                for i in range(CP):
                    k = ph + (qbase + i * RH_DIM + d)
                    v.append(plsc.load_gather(
                        small_g, [lax.shift_right_logical(k, 7), rows,
                                  k & 127]))
                for t in range(BLUR):
                    acc = (v[0] * c(0, t) + v[1] * c(1, t)
                           + v[2] * c(2, t) + v[3] * c(3, t))
                    plsc.store_scatter(
                        small_buf, [(iota & 7) * BLUR + t,
                                    jnp.full((16,), d, jnp.int32)], acc)
            pltpu.sync_copy(small_buf,
                            out_ref.at[pl.ds((base + b0) * BLUR, 8 * BLUR)])
            return carry
        lax.fori_loop(0, BPW // 8, sbc_body, 0)

    small_pass(rh_out, 0)
    small_pass(th_out, SROW)


def _run(indices, coefF, poses_w, small2):
    f32 = jnp.float32
    kern = pl.kernel(
        _blend_kernel,
        out_type=[
            jax.ShapeDtypeStruct((BT, POSE_DIM), f32),
            jax.ShapeDtypeStruct((BT, RH_DIM), f32),
            jax.ShapeDtypeStruct((BT, RH_DIM), f32),
        ],
        mesh=plsc.VectorSubcoreMesh(core_axis_name="c", subcore_axis_name="s",
                                    num_cores=NC, num_subcores=NS),
        compiler_params=pltpu.CompilerParams(use_tc_tiling_on_sc=True,
                                             needs_layout_passes=False),
        scratch_types=[
            pltpu.VMEM((BPW,), jnp.int32),
            pltpu.VMEM((2, BPW), jnp.int32),
            pltpu.VMEM((2, BPW, 128), f32),
            pltpu.VMEM((BPW, 128), f32),
            pltpu.VMEM((2, BPW, 128), f32),
            pltpu.VMEM((CP * BLUR * 16,), f32),
            pltpu.VMEM((8 * BLUR, POSE_DIM), f32),
            pltpu.VMEM((8 * BLUR, RH_DIM), f32),
            pltpu.SemaphoreType.DMA,
            pltpu.SemaphoreType.DMA,
        ],
    )
    return kern(indices, coefF, poses_w, small2)


def kernel(indices, blur_num, shapes_w, poses_w, Rhs_w, Ths_w, M):
    f32 = jnp.float32
    indices = indices.astype(jnp.int32)
    t = jnp.arange(BLUR, dtype=f32) / (jnp.asarray(blur_num, f32) - 1.0)
    t = jnp.where(t == 0.0, t + 1e-06, t)
    t = jnp.where(t == 1.0, t - 1e-06, t)
    tm = jnp.stack([jnp.ones_like(t), t, t ** 2, t ** 3], axis=-1)
    coeffs = tm @ M.astype(f32)
    coefF = jnp.broadcast_to(coeffs.T.reshape(CP * BLUR, 1),
                             (CP * BLUR, 16)).reshape(CP * BLUR * 16)

    small2 = jnp.concatenate(
        [Rhs_w.astype(f32), Ths_w.astype(f32)], axis=1)
    small2 = small2.reshape(SMALL_VROWS, 128)

    pose2, rh2, th2 = _run(indices, coefF, poses_w.astype(f32), small2)
    pose_t = pose2.reshape(BATCH, BLUR, POSE_DIM)
    rh_t = rh2.reshape(BATCH, BLUR, RH_DIM)
    th_t = th2.reshape(BATCH, BLUR, RH_DIM)
    shape_t = jnp.broadcast_to(shapes_w.reshape(1, 1, 10).astype(f32),
                               (BATCH, BLUR, 10))
    return (shape_t, pose_t, rh_t, th_t)
